# Initial kernel scaffold; baseline (speedup 1.0000x reference)
import jax
import jax.numpy as jnp
from jax.experimental import pallas as pl


def kernel(indices, items_emb):
    raise NotImplementedError("write your pallas kernel here")


# speedup vs baseline: 1.0936x; 1.0936x over previous
import functools

import jax
import jax.numpy as jnp
from jax import lax
from jax.experimental import pallas as pl
from jax.experimental.pallas import tpu as pltpu
from jax.experimental.pallas import tpu_sc as plsc

NC = 2
NS = 16
NW = NC * NS

IDX_W = 128
BLK = 1024
ROWS_PER_BLK = BLK // IDX_W


def _make_gather(n_total: int, dim: int):
  assert n_total % (NW * BLK) == 0
  per_w = n_total // NW
  n_blk = per_w // BLK
  idx_rows_per_w = per_w // IDX_W

  mesh = plsc.VectorSubcoreMesh(core_axis_name="c", subcore_axis_name="s")

  @functools.partial(
      pl.kernel,
      mesh=mesh,
      compiler_params=pltpu.CompilerParams(use_tc_tiling_on_sc=False),
      out_type=jax.ShapeDtypeStruct((n_total, dim), jnp.float32),
      scratch_types=[
          pltpu.VMEM((ROWS_PER_BLK, IDX_W), jnp.int32),
          pltpu.VMEM((BLK, dim), jnp.float32),
          pltpu.SemaphoreType.DMA,
      ],
  )
  def gather_kernel(idx_hbm, table_hbm, out_hbm, idx_v, rows_v, sem):
    wid = lax.axis_index("s") * NC + lax.axis_index("c")
    idx_row0 = wid * idx_rows_per_w
    out0 = wid * per_w

    def body(i, carry):
      pltpu.sync_copy(
          idx_hbm.at[pl.ds(idx_row0 + i * ROWS_PER_BLK, ROWS_PER_BLK), :],
          idx_v)
      copies = []
      for j in range(ROWS_PER_BLK):
        copies.append(
            pltpu.async_copy(
                table_hbm.at[idx_v.at[j]],
                rows_v.at[pl.ds(j * IDX_W, IDX_W), :],
                sem))
      for c in copies:
        c.wait()
      pltpu.sync_copy(rows_v, out_hbm.at[pl.ds(out0 + i * BLK, BLK), :])
      return carry

    lax.fori_loop(0, n_blk, body, 0)

  return gather_kernel


def kernel(indices, items_emb):
  b, s = indices.shape
  m, dim = items_emb.shape
  n_total = b * s
  idx2d = indices.reshape(n_total // IDX_W, IDX_W).astype(jnp.int32)
  out = _make_gather(n_total, dim)(idx2d, items_emb)
  return out.reshape(b, s, dim)


# speedup vs baseline: 1.1075x; 1.0128x over previous
import functools

import jax
import jax.numpy as jnp
from jax import lax
from jax.experimental import pallas as pl
from jax.experimental.pallas import tpu as pltpu
from jax.experimental.pallas import tpu_sc as plsc

NC = 2
NS = 16
NW = NC * NS

IDX_W = 128
BLK = 1280
RPB = BLK // IDX_W


def _make_gather(n_total: int, dim: int):
  assert n_total % (NW * BLK) == 0
  per_w = n_total // NW
  n_blk = per_w // BLK
  assert n_blk % 2 == 0 and n_blk >= 4
  idx_rows_per_w = per_w // IDX_W

  mesh = plsc.VectorSubcoreMesh(core_axis_name="c", subcore_axis_name="s")

  @functools.partial(
      pl.kernel,
      mesh=mesh,
      compiler_params=pltpu.CompilerParams(use_tc_tiling_on_sc=False),
      out_type=jax.ShapeDtypeStruct((n_total, dim), jnp.float32),
      scratch_types=[
          pltpu.VMEM((2, RPB, IDX_W), jnp.int32),
          pltpu.VMEM((2, BLK, dim), jnp.float32),
          pltpu.SemaphoreType.DMA,
          pltpu.SemaphoreType.DMA,
          pltpu.SemaphoreType.DMA,
          pltpu.SemaphoreType.DMA,
      ],
  )
  def gather_kernel(idx_hbm, table_hbm, out_hbm, idx_v, rows_v,
                    gsem0, gsem1, osem0, osem1):
    wid = lax.axis_index("s") * NC + lax.axis_index("c")
    idx_row0 = wid * idx_rows_per_w
    out0 = wid * per_w
    gsem = (gsem0, gsem1)
    osem = (osem0, osem1)

    def fire_gathers(j, b):
      pltpu.sync_copy(
          idx_hbm.at[pl.ds(idx_row0 + j * RPB, RPB), :], idx_v.at[b])
      for r in range(RPB):
        pltpu.async_copy(
            table_hbm.at[idx_v.at[b].at[r]],
            rows_v.at[b].at[pl.ds(r * IDX_W, IDX_W), :],
            gsem[b])

    def drain_gathers(b):
      pltpu.make_async_copy(
          table_hbm.at[pl.ds(0, BLK), :], rows_v.at[b], gsem[b]).wait()

    def fire_writeback(j, b):
      pltpu.async_copy(
          rows_v.at[b], out_hbm.at[pl.ds(out0 + j * BLK, BLK), :], osem[b])

    def drain_writeback(b):
      pltpu.make_async_copy(
          rows_v.at[b], out_hbm.at[pl.ds(out0, BLK), :], osem[b]).wait()

    fire_gathers(0, 0)
    fire_gathers(1, 1)
    drain_gathers(0)
    fire_writeback(0, 0)

    @pl.loop(2, n_blk, step=2)
    def _steady(i):
      for b in range(2):
        j = i + b
        drain_writeback(b)
        fire_gathers(j, b)
        drain_gathers(1 - b)
        fire_writeback(j - 1, 1 - b)

    drain_gathers(1)
    fire_writeback(n_blk - 1, 1)
    drain_writeback(0)
    drain_writeback(1)

  return gather_kernel


def kernel(indices, items_emb):
  b, s = indices.shape
  m, dim = items_emb.shape
  n_total = b * s
  idx2d = indices.reshape(n_total // IDX_W, IDX_W).astype(jnp.int32)
  out = _make_gather(n_total, dim)(idx2d, items_emb)
  return out.reshape(b, s, dim)


# speedup vs baseline: 1.8040x; 1.6289x over previous
import functools

import jax
import jax.numpy as jnp
from jax import lax
from jax.experimental import pallas as pl
from jax.experimental.pallas import tpu as pltpu
from jax.experimental.pallas import tpu_sc as plsc

NC = 2
NS = 16
NW = NC * NS

RBLK = 16


def _make_gather(n_rows: int, n_cols: int, dim: int):
  assert n_rows % (NW * RBLK) == 0
  rows_per_w = n_rows // NW
  n_blk = rows_per_w // RBLK
  assert n_blk % 2 == 0 and n_blk >= 4

  mesh = plsc.VectorSubcoreMesh(core_axis_name="c", subcore_axis_name="s")

  @functools.partial(
      pl.kernel,
      mesh=mesh,
      compiler_params=pltpu.CompilerParams(use_tc_tiling_on_sc=False),
      out_type=jax.ShapeDtypeStruct((n_rows, n_cols, dim), jnp.float32),
      scratch_types=[
          pltpu.VMEM((rows_per_w, n_cols), jnp.int32),
          pltpu.VMEM((2, RBLK, n_cols, dim), jnp.float32),
          pltpu.SemaphoreType.DMA,
          pltpu.SemaphoreType.DMA,
          pltpu.SemaphoreType.DMA,
          pltpu.SemaphoreType.DMA,
      ],
  )
  def gather_kernel(idx_hbm, table_hbm, out_hbm, idx_v, rows_v,
                    gsem0, gsem1, osem0, osem1):
    wid = lax.axis_index("s") * NC + lax.axis_index("c")
    row0 = wid * rows_per_w
    gsem = (gsem0, gsem1)
    osem = (osem0, osem1)

    pltpu.sync_copy(idx_hbm.at[pl.ds(row0, rows_per_w), :], idx_v)

    def fire_gather(j, b):
      for r in range(RBLK):
        pltpu.async_copy(
            table_hbm.at[idx_v.at[j * RBLK + r]],
            rows_v.at[b].at[r], gsem[b])

    def drain_gather(b):
      pltpu.make_async_copy(
          out_hbm.at[pl.ds(row0, RBLK), :, :], rows_v.at[b], gsem[b]).wait()

    def fire_writeback(j, b):
      pltpu.async_copy(
          rows_v.at[b], out_hbm.at[pl.ds(row0 + j * RBLK, RBLK), :, :],
          osem[b])

    def drain_writeback(b):
      pltpu.make_async_copy(
          rows_v.at[b], out_hbm.at[pl.ds(row0, RBLK), :, :], osem[b]).wait()

    fire_gather(0, 0)
    fire_gather(1, 1)
    drain_gather(0)
    fire_writeback(0, 0)

    @pl.loop(2, n_blk, step=2)
    def _steady(i):
      for b in range(2):
        j = i + b
        drain_writeback(b)
        fire_gather(j, b)
        drain_gather(1 - b)
        fire_writeback(j - 1, 1 - b)

    drain_gather(1)
    fire_writeback(n_blk - 1, 1)
    drain_writeback(0)
    drain_writeback(1)

  return gather_kernel


def kernel(indices, items_emb):
  n_rows, n_cols = indices.shape
  m, dim = items_emb.shape
  return _make_gather(n_rows, n_cols, dim)(
      indices.astype(jnp.int32), items_emb)
